# Initial kernel scaffold; baseline (speedup 1.0000x reference)
import jax
import jax.numpy as jnp
from jax.experimental import pallas as pl


def kernel(table, idx):
    raise NotImplementedError("write your pallas kernel here")


# speedup vs baseline: 150.4967x; 150.4967x over previous
import jax
import jax.numpy as jnp
from jax.experimental import pallas as pl
from jax.experimental.pallas import tpu as pltpu

_TB = 1024
_U = 16


def _round_up(x: int, m: int) -> int:
    return ((x + m - 1) // m) * m


def _gather_kernel(idx_ref, table_ref, out_ref):
    tb = out_ref.shape[0]
    base = pl.program_id(0) * tb

    def chunk(c, carry):
        cb = c * _U
        for j in range(_U):
            t = idx_ref[base + cb + j]
            out_ref[pl.ds(cb + j, 1), :, :] = table_ref[pl.ds(t, 1), :, :]
        return carry

    jax.lax.fori_loop(0, tb // _U, chunk, 0)


def kernel(table, idx):
    V, D = table.shape
    out_shape = idx.shape + (D,)
    idx_flat = idx.reshape(-1).astype(jnp.int32)
    N = int(idx_flat.shape[0])

    tb = _round_up(min(_TB, N), _U)
    n_pad = _round_up(N, tb)
    if n_pad != N:
        idx_flat = jnp.pad(idx_flat, (0, n_pad - N))
    n_blocks = n_pad // tb

    table3 = table.reshape(V, 1, D)

    out = pl.pallas_call(
        _gather_kernel,
        out_shape=jax.ShapeDtypeStruct((n_pad, 1, D), table.dtype),
        grid=(n_blocks,),
        in_specs=[
            pl.BlockSpec(memory_space=pltpu.SMEM),
            pl.BlockSpec((V, 1, D), lambda i: (0, 0, 0)),
        ],
        out_specs=pl.BlockSpec((tb, 1, D), lambda i: (i, 0, 0)),
        compiler_params=pltpu.CompilerParams(
            dimension_semantics=("parallel",),
            vmem_limit_bytes=60 << 20,
        ),
    )(idx_flat, table3)

    return out[:N].reshape(out_shape)


# speedup vs baseline: 241.3238x; 1.6035x over previous
import jax
import jax.numpy as jnp
from jax.experimental import pallas as pl
from jax.experimental.pallas import tpu as pltpu

_TB = 1024
_G = 8
_U = 2
_NC = 1


def _round_up(x: int, m: int) -> int:
    return ((x + m - 1) // m) * m


def _gather_kernel(idx_ref, table_ref, out_ref):
    tb = out_ref.shape[0]
    nb_per_core = pl.num_programs(1)
    base = (pl.program_id(0) * nb_per_core + pl.program_id(1)) * tb

    def chunk(c, carry):
        for u in range(_U):
            g = (c * _U + u) * _G
            rows = []
            for j in range(_G):
                t = idx_ref[base + g + j]
                rows.append(table_ref[t, 0, :])
            out_ref[pl.ds(pl.multiple_of(g, _G), _G), :] = jnp.stack(rows, axis=0)
        return carry

    jax.lax.fori_loop(0, tb // (_G * _U), chunk, 0)


def kernel(table, idx):
    V, D = table.shape
    out_shape = idx.shape + (D,)
    idx_flat = idx.reshape(-1).astype(jnp.int32)
    N = int(idx_flat.shape[0])

    tb = _round_up(min(_TB, N), _G * _U)
    n_pad = _round_up(N, tb * _NC)
    if n_pad != N:
        idx_flat = jnp.pad(idx_flat, (0, n_pad - N))
    n_blocks = n_pad // tb
    nb_per_core = n_blocks // _NC

    table3 = table.reshape(V, 1, D)

    out = pl.pallas_call(
        _gather_kernel,
        out_shape=jax.ShapeDtypeStruct((n_pad, D), table.dtype),
        grid=(_NC, nb_per_core),
        in_specs=[
            pl.BlockSpec(memory_space=pltpu.SMEM),
            pl.BlockSpec((V, 1, D), lambda c, b: (0, 0, 0)),
        ],
        out_specs=pl.BlockSpec((tb, D), lambda c, b: (c * (n_pad // (tb * _NC)) + b, 0)),
        compiler_params=pltpu.CompilerParams(
            dimension_semantics=("core_parallel", "arbitrary"),
            vmem_limit_bytes=60 << 20,
        ),
    )(idx_flat, table3)

    return out[:N].reshape(out_shape)


# speedup vs baseline: 245.3213x; 1.0166x over previous
import jax
import jax.numpy as jnp
from jax.experimental import pallas as pl
from jax.experimental.pallas import tpu as pltpu

_TB = 1024
_G = 8
_U = 2
_NC = 1


def _round_up(x: int, m: int) -> int:
    return ((x + m - 1) // m) * m


def _gather_kernel(idx_ref, table_ref, out_ref):
    tb = out_ref.shape[0]
    nb_per_core = pl.num_programs(1)
    base = (pl.program_id(0) * nb_per_core + pl.program_id(1)) * tb

    def chunk(c, carry):
        for u in range(_U):
            g = (c * _U + u) * _G
            rows = []
            for j in range(_G):
                t = idx_ref[base + g + j]
                rows.append(table_ref[pl.ds(t, 1), :, :])
            tile = jnp.concatenate(rows, axis=0)
            out_ref[pl.ds(pl.multiple_of(g, _G), _G), :] = tile[:, 0, :]
        return carry

    jax.lax.fori_loop(0, tb // (_G * _U), chunk, 0)


def kernel(table, idx):
    V, D = table.shape
    out_shape = idx.shape + (D,)
    idx_flat = idx.reshape(-1).astype(jnp.int32)
    N = int(idx_flat.shape[0])

    tb = _round_up(min(_TB, N), _G * _U)
    n_pad = _round_up(N, tb * _NC)
    if n_pad != N:
        idx_flat = jnp.pad(idx_flat, (0, n_pad - N))
    n_blocks = n_pad // tb
    nb_per_core = n_blocks // _NC

    table3 = table.reshape(V, 1, D)

    out = pl.pallas_call(
        _gather_kernel,
        out_shape=jax.ShapeDtypeStruct((n_pad, D), table.dtype),
        grid=(_NC, nb_per_core),
        in_specs=[
            pl.BlockSpec(memory_space=pltpu.SMEM),
            pl.BlockSpec((V, 1, D), lambda c, b: (0, 0, 0)),
        ],
        out_specs=pl.BlockSpec((tb, D), lambda c, b: (c * (n_pad // (tb * _NC)) + b, 0)),
        compiler_params=pltpu.CompilerParams(
            dimension_semantics=("core_parallel", "arbitrary"),
            vmem_limit_bytes=60 << 20,
        ),
    )(idx_flat, table3)

    return out[:N].reshape(out_shape)


# speedup vs baseline: 265.4207x; 1.0819x over previous
import jax
import jax.numpy as jnp
from jax.experimental import pallas as pl
from jax.experimental.pallas import tpu as pltpu

_TB = 1024
_G = 8
_U = 4
_NC = 1


def _round_up(x: int, m: int) -> int:
    return ((x + m - 1) // m) * m


def _gather_kernel(idx_ref, table_ref, out_ref):
    tb = out_ref.shape[0]
    nb_per_core = pl.num_programs(1)
    base = (pl.program_id(0) * nb_per_core + pl.program_id(1)) * tb

    def chunk(c, carry):
        for u in range(_U):
            g = (c * _U + u) * _G
            rows = []
            for j in range(_G):
                t = idx_ref[base + g + j]
                rows.append(table_ref[pl.ds(t, 1), :, :])
            tile = jnp.concatenate(rows, axis=0)
            out_ref[pl.ds(pl.multiple_of(g, _G), _G), :] = tile[:, 0, :]
        return carry

    jax.lax.fori_loop(0, tb // (_G * _U), chunk, 0)


def kernel(table, idx):
    V, D = table.shape
    out_shape = idx.shape + (D,)
    idx_flat = idx.reshape(-1).astype(jnp.int32)
    N = int(idx_flat.shape[0])

    tb = _round_up(min(_TB, N), _G * _U)
    n_pad = _round_up(N, tb * _NC)
    if n_pad != N:
        idx_flat = jnp.pad(idx_flat, (0, n_pad - N))
    n_blocks = n_pad // tb
    nb_per_core = n_blocks // _NC

    table3 = table.reshape(V, 1, D)

    out = pl.pallas_call(
        _gather_kernel,
        out_shape=jax.ShapeDtypeStruct((n_pad, D), table.dtype),
        grid=(_NC, nb_per_core),
        in_specs=[
            pl.BlockSpec(memory_space=pltpu.SMEM),
            pl.BlockSpec((V, 1, D), lambda c, b: (0, 0, 0)),
        ],
        out_specs=pl.BlockSpec((tb, D), lambda c, b: (c * (n_pad // (tb * _NC)) + b, 0)),
        compiler_params=pltpu.CompilerParams(
            dimension_semantics=("core_parallel", "arbitrary"),
            vmem_limit_bytes=60 << 20,
        ),
    )(idx_flat, table3)

    return out[:N].reshape(out_shape)
